# baseline (speedup 1.0000x reference)
import functools

import jax
import jax.numpy as jnp
from jax import lax
from jax.experimental import pallas as pl
from jax.experimental.pallas import tpu as pltpu
from jax.experimental.pallas import tpu_sc as plsc

E_ROWS = 150000
IN_DIM = 512
OUT_DIM = 64
PAD_DIM = 128
B = 100000


_MM_ROWS = 6000


def _mm_body(x_ref, wt_ref, b_ref, o_ref):
    o_ref[...] = (
        jnp.dot(x_ref[...], wt_ref[...], preferred_element_type=jnp.float32)
        + b_ref[...]
    )


def _transform_table(x, wt, b2d):
    return pl.pallas_call(
        _mm_body,
        grid=(E_ROWS // _MM_ROWS,),
        in_specs=[
            pl.BlockSpec((_MM_ROWS, IN_DIM), lambda i: (i, 0)),
            pl.BlockSpec((IN_DIM, PAD_DIM), lambda i: (0, 0)),
            pl.BlockSpec((1, PAD_DIM), lambda i: (0, 0)),
        ],
        out_specs=pl.BlockSpec((_MM_ROWS, PAD_DIM), lambda i: (i, 0)),
        out_shape=jax.ShapeDtypeStruct((E_ROWS, PAD_DIM), jnp.float32),
    )(x, wt, b2d)


_B_PAD = 102400
_PER_W = _B_PAD // 32
_CHUNK = 320
_NCH = _PER_W // _CHUNK
_DEPTH = 3


def _gather_body(table_hbm, idx_hbm, out_hbm, i0, i1, i2, r0, r1, r2, s0, s1, s2):
    wid = lax.axis_index("s") * 2 + lax.axis_index("c")
    base = wid * _PER_W
    ibufs, rbufs, sems = [i0, i1, i2], [r0, r1, r2], [s0, s1, s2]
    hs = [None] * _NCH
    for j in range(_DEPTH - 1):
        pltpu.sync_copy(idx_hbm.at[pl.ds(base + j * _CHUNK, _CHUNK)], ibufs[j])
        hs[j] = pltpu.async_copy(table_hbm.at[ibufs[j]], rbufs[j], sems[j])
    for k in range(_NCH):
        kk = k + _DEPTH - 1
        if kk < _NCH:
            s = kk % _DEPTH
            pltpu.sync_copy(
                idx_hbm.at[pl.ds(base + kk * _CHUNK, _CHUNK)], ibufs[s]
            )
            hs[kk] = pltpu.async_copy(table_hbm.at[ibufs[s]], rbufs[s], sems[s])
        hs[k].wait()
        pltpu.sync_copy(
            rbufs[k % _DEPTH], out_hbm.at[pl.ds(base + k * _CHUNK, _CHUNK)]
        )


def _gather_rows(table, idx_pad):
    mesh = plsc.VectorSubcoreMesh(core_axis_name="c", subcore_axis_name="s")
    k = functools.partial(
        pl.kernel,
        mesh=mesh,
        out_type=jax.ShapeDtypeStruct((_B_PAD, PAD_DIM), jnp.float32),
        scratch_types=[
            pltpu.VMEM((_CHUNK,), jnp.int32),
            pltpu.VMEM((_CHUNK,), jnp.int32),
            pltpu.VMEM((_CHUNK,), jnp.int32),
            pltpu.VMEM((_CHUNK, PAD_DIM), jnp.float32),
            pltpu.VMEM((_CHUNK, PAD_DIM), jnp.float32),
            pltpu.VMEM((_CHUNK, PAD_DIM), jnp.float32),
            pltpu.SemaphoreType.DMA,
            pltpu.SemaphoreType.DMA,
            pltpu.SemaphoreType.DMA,
        ],
    )(_gather_body)
    return k(table, idx_pad)


def kernel(edge_features, neighbors_edge_idxs, W, b):
    wt_pad = jnp.concatenate(
        [W.T, jnp.zeros((IN_DIM, PAD_DIM - OUT_DIM), jnp.float32)], axis=1
    )
    b_pad = jnp.concatenate(
        [b, jnp.zeros((PAD_DIM - OUT_DIM,), jnp.float32)]
    ).reshape(1, PAD_DIM)
    table = _transform_table(edge_features, wt_pad, b_pad)
    idx = neighbors_edge_idxs.astype(jnp.int32)
    idx_pad = jnp.concatenate([idx, jnp.zeros((_B_PAD - B,), jnp.int32)])
    out = _gather_rows(table, idx_pad)
    return out[:B, :OUT_DIM]
